# baseline (speedup 1.0000x reference)
import functools

import jax
import jax.numpy as jnp
from jax.experimental import pallas as pl
from jax.experimental.pallas import tpu as pltpu


def _body(mb_ref, nh_ref, nf_ref, w1a_t_ref, w1b_t_ref, w1c_t_ref,
          w2_t_ref, b1_ref, b2_ref, out_ref):
    edge = jnp.sum(mb_ref[...], axis=1)
    acc = jnp.dot(nh_ref[...], w1a_t_ref[...],
                  preferred_element_type=jnp.float32)
    acc += jnp.dot(edge, w1c_t_ref[...], preferred_element_type=jnp.float32)
    acc += jnp.dot(nf_ref[...], w1b_t_ref[...],
                   preferred_element_type=jnp.float32)
    h = jnp.maximum(acc + b1_ref[...], 0.0)
    out = jnp.dot(h, w2_t_ref[...], preferred_element_type=jnp.float32)
    out_ref[...] = jnp.maximum(out + b2_ref[...], 0.0)


@functools.partial(jax.jit, static_argnames=("block_rows", "interpret"))
def _run(mb, nh, nf, W1, b1, W2, b2, block_rows=1000, interpret=False):
    n, deg, emb = mb.shape
    nodemb = nh.shape[1]
    nfeat = nf.shape[1]
    hidden = W1.shape[0]
    out_dim = W2.shape[0]

    w1a_t = W1[:, :nodemb].T
    w1b_t = W1[:, nodemb:nodemb + nfeat].T
    w1c_t = W1[:, nodemb + nfeat:].T
    w2_t = W2.T
    b1r = b1.reshape(1, hidden)
    b2r = b2.reshape(1, out_dim)

    grid = n // block_rows
    return pl.pallas_call(
        _body,
        grid=(grid,),
        in_specs=[
            pl.BlockSpec((block_rows, deg, emb), lambda i: (i, 0, 0)),
            pl.BlockSpec((block_rows, nodemb), lambda i: (i, 0)),
            pl.BlockSpec((block_rows, nfeat), lambda i: (i, 0)),
            pl.BlockSpec((nodemb, hidden), lambda i: (0, 0)),
            pl.BlockSpec((nfeat, hidden), lambda i: (0, 0)),
            pl.BlockSpec((emb, hidden), lambda i: (0, 0)),
            pl.BlockSpec((hidden, out_dim), lambda i: (0, 0)),
            pl.BlockSpec((1, hidden), lambda i: (0, 0)),
            pl.BlockSpec((1, out_dim), lambda i: (0, 0)),
        ],
        out_specs=pl.BlockSpec((block_rows, out_dim), lambda i: (i, 0)),
        out_shape=jax.ShapeDtypeStruct((n, out_dim), jnp.float32),
        compiler_params=pltpu.CompilerParams(
            dimension_semantics=("arbitrary",)),
        interpret=interpret,
    )(mb, nh, nf, w1a_t, w1b_t, w1c_t, w2_t, b1r, b2r)


def kernel(mailbox_edge_hidden_representation, node_hidden_state,
           node_features, W1, b1, W2, b2):
    return _run(mailbox_edge_hidden_representation, node_hidden_state,
                node_features, W1, b1, W2, b2)


# speedup vs baseline: 9.2186x; 9.2186x over previous
import functools

import jax
import jax.numpy as jnp
from jax.experimental import pallas as pl
from jax.experimental.pallas import tpu as pltpu


def _body(mb_ref, nh_ref, nf_ref, w1a_ref, w1b_ref, w1c_ref, w2_ref,
          b1_ref, b2_ref, out_ref):
    m = mb_ref[...]
    a = m[:256] + m[256:]
    a = a[:128] + a[128:]
    a = a[:64] + a[64:]
    a = a[:32] + a[32:]
    e = a[:16] + a[16:]
    acc = jnp.dot(w1a_ref[...], nh_ref[...],
                  preferred_element_type=jnp.float32)
    acc += jnp.dot(w1c_ref[...], e, preferred_element_type=jnp.float32)
    acc += jnp.dot(w1b_ref[...], nf_ref[...],
                   preferred_element_type=jnp.float32)
    h = jnp.maximum(acc + b1_ref[...], 0.0)
    o = jnp.dot(w2_ref[...], h, preferred_element_type=jnp.float32)
    o = jnp.maximum(o + b2_ref[...], 0.0)
    out_ref[...] = o.T


@functools.partial(jax.jit, static_argnames=("block_cols", "interpret"))
def _run(mb, nh, nf, W1, b1, W2, b2, block_cols=4096, interpret=False):
    n, deg, emb = mb.shape
    nodemb = nh.shape[1]
    nfeat = nf.shape[1]
    hidden = W1.shape[0]
    out_dim = W2.shape[0]
    k_mb = deg * emb

    mbT = jnp.transpose(mb, (1, 2, 0)).reshape(k_mb, n)
    nhT = nh.T
    nfT = nf.T
    w1a = W1[:, :nodemb]
    w1b = W1[:, nodemb:nodemb + nfeat]
    w1c = W1[:, nodemb + nfeat:]
    b1c = b1.reshape(hidden, 1)
    b2c = b2.reshape(out_dim, 1)

    B = block_cols
    return pl.pallas_call(
        _body,
        grid=(pl.cdiv(n, B),),
        in_specs=[
            pl.BlockSpec((k_mb, B), lambda i: (0, i)),
            pl.BlockSpec((nodemb, B), lambda i: (0, i)),
            pl.BlockSpec((nfeat, B), lambda i: (0, i)),
            pl.BlockSpec((hidden, nodemb), lambda i: (0, 0)),
            pl.BlockSpec((hidden, nfeat), lambda i: (0, 0)),
            pl.BlockSpec((hidden, emb), lambda i: (0, 0)),
            pl.BlockSpec((out_dim, hidden), lambda i: (0, 0)),
            pl.BlockSpec((hidden, 1), lambda i: (0, 0)),
            pl.BlockSpec((out_dim, 1), lambda i: (0, 0)),
        ],
        out_specs=pl.BlockSpec((B, out_dim), lambda i: (i, 0)),
        out_shape=jax.ShapeDtypeStruct((n, out_dim), jnp.float32),
        compiler_params=pltpu.CompilerParams(
            dimension_semantics=("arbitrary",)),
        interpret=interpret,
    )(mbT, nhT, nfT, w1a, w1b, w1c, W2, b1c, b2c)


def kernel(mailbox_edge_hidden_representation, node_hidden_state,
           node_features, W1, b1, W2, b2):
    return _run(mailbox_edge_hidden_representation, node_hidden_state,
                node_features, W1, b1, W2, b2)


# speedup vs baseline: 9.4731x; 1.0276x over previous
import functools

import jax
import jax.numpy as jnp
from jax.experimental import pallas as pl
from jax.experimental.pallas import tpu as pltpu


def _body(mb_ref, nh_ref, nf_ref, w1a_ref, w1b_ref, w1c_ref, w2_ref,
          b1_ref, b2_ref, out_ref):
    e = mb_ref[0:16, :]
    for k in range(1, 32):
        e = e + mb_ref[16 * k:16 * (k + 1), :]
    acc = jnp.dot(w1a_ref[...], nh_ref[...],
                  preferred_element_type=jnp.float32)
    acc += jnp.dot(w1c_ref[...], e, preferred_element_type=jnp.float32)
    acc += jnp.dot(w1b_ref[...], nf_ref[...],
                   preferred_element_type=jnp.float32)
    h = jnp.maximum(acc + b1_ref[...], 0.0)
    o = jax.lax.dot_general(h, w2_ref[...], (((0,), (1,)), ((), ())),
                            preferred_element_type=jnp.float32)
    out_ref[...] = jnp.maximum(o + b2_ref[...].T, 0.0)


@functools.partial(jax.jit, static_argnames=("block_cols", "interpret"))
def _run(mb, nh, nf, W1, b1, W2, b2, block_cols=4096, interpret=False):
    n, deg, emb = mb.shape
    nodemb = nh.shape[1]
    nfeat = nf.shape[1]
    hidden = W1.shape[0]
    out_dim = W2.shape[0]
    k_mb = deg * emb

    mbT = jnp.transpose(mb, (1, 2, 0)).reshape(k_mb, n)
    nhT = nh.T
    nfT = nf.T
    w1a = W1[:, :nodemb]
    w1b = W1[:, nodemb:nodemb + nfeat]
    w1c = W1[:, nodemb + nfeat:]
    b1c = b1.reshape(hidden, 1)
    b2c = b2.reshape(out_dim, 1)

    B = block_cols
    return pl.pallas_call(
        _body,
        grid=(pl.cdiv(n, B),),
        in_specs=[
            pl.BlockSpec((k_mb, B), lambda i: (0, i)),
            pl.BlockSpec((nodemb, B), lambda i: (0, i)),
            pl.BlockSpec((nfeat, B), lambda i: (0, i)),
            pl.BlockSpec((hidden, nodemb), lambda i: (0, 0)),
            pl.BlockSpec((hidden, nfeat), lambda i: (0, 0)),
            pl.BlockSpec((hidden, emb), lambda i: (0, 0)),
            pl.BlockSpec((out_dim, hidden), lambda i: (0, 0)),
            pl.BlockSpec((hidden, 1), lambda i: (0, 0)),
            pl.BlockSpec((out_dim, 1), lambda i: (0, 0)),
        ],
        out_specs=pl.BlockSpec((B, out_dim), lambda i: (i, 0)),
        out_shape=jax.ShapeDtypeStruct((n, out_dim), jnp.float32),
        compiler_params=pltpu.CompilerParams(
            dimension_semantics=("arbitrary",)),
        interpret=interpret,
    )(mbT, nhT, nfT, w1a, w1b, w1c, W2, b1c, b2c)


def kernel(mailbox_edge_hidden_representation, node_hidden_state,
           node_features, W1, b1, W2, b2):
    return _run(mailbox_edge_hidden_representation, node_hidden_state,
                node_features, W1, b1, W2, b2)
